# baseline (speedup 1.0000x reference)
import jax
import jax.numpy as jnp
from jax import lax
from jax.experimental import pallas as pl
from jax.experimental.pallas import tpu as pltpu
from jax.experimental.pallas import tpu_sc as plsc

K = 256
G = 2048
X0 = -8.0
X1 = 8.0
H = (X1 - X0) / (G - 1)
INV_H = 1.0 / H
GR = G // 128
KU = 15

NC, NS, L = 2, 16, 16
NW = NC * NS


def _tab_body(lev_ref, beta_ref, tab_ref):
    beta = beta_ref[0]
    l0 = lev_ref[0]
    gidx = (lax.broadcasted_iota(jnp.int32, (GR, 128), 0) * 128
            + lax.broadcasted_iota(jnp.int32, (GR, 128), 1))
    xg = X0 + H * gidx.astype(jnp.float32)

    def body(j, acc):
        for u in range(KU):
            k = j * KU + u
            lk = lev_ref[k]
            lk1 = lev_ref[k + 1]
            w = lk1 - lk
            b = 0.5 * (lk1 + lk)
            acc = acc + w * jax.nn.sigmoid(beta * (xg - b))
        return acc

    init = jnp.full((GR, 128), l0, jnp.float32)
    tab_ref[...] = lax.fori_loop(0, (K - 1) // KU, body, init)


def _make_sc_lookup(b_dim, r_dim, c_dim):
    rows_w = (b_dim * r_dim) // NW
    cv = c_dim // L
    VU = 4
    qpr = cv // VU
    w_per_b = r_dim // rows_w
    mesh = plsc.VectorSubcoreMesh(core_axis_name="c", subcore_axis_name="s",
                                  num_cores=NC, num_subcores=NS)
    hr = (rows_w // 2 + 7) // 8 * 8
    hr2 = rows_w - hr

    def _sc_body(x_hbm, tab_hbm, out_hbm, tab_v, x_v, y_v, xsem0, xsem1, ysem):
        wid = lax.axis_index("c") * NS + lax.axis_index("s")
        b = wid // w_per_b
        r0 = (wid % w_per_b) * rows_w

        xcopy0 = pltpu.make_async_copy(
            x_hbm.at[b, pl.ds(r0, hr), :], x_v.at[pl.ds(0, hr), :], xsem0)
        xcopy0.start()
        xcopy1 = pltpu.make_async_copy(
            x_hbm.at[b, pl.ds(r0 + hr, hr2), :], x_v.at[pl.ds(hr, hr2), :], xsem1)
        xcopy1.start()
        pltpu.sync_copy(tab_hbm, tab_v)

        def make_body(lo):
            def body(q):
                r = lo + q // qpr
                c0 = (q % qpr) * (VU * L)
                for j in range(VU):
                    o = c0 + j * L
                    xv = x_v[r, pl.ds(o, L)]
                    t = (jnp.clip(xv, X0, X1) - X0) * INV_H
                    idx = jnp.minimum(t.astype(jnp.int32), G - 2)
                    fr = t - idx.astype(jnp.float32)
                    y0 = plsc.load_gather(tab_v, [idx])
                    y1 = plsc.load_gather(tab_v, [idx + 1])
                    y_v[r, pl.ds(o, L)] = y0 + fr * (y1 - y0)
            return body

        xcopy0.wait()
        plsc.parallel_loop(0, hr * qpr, unroll=4)(make_body(0))
        ycopy0 = pltpu.make_async_copy(
            y_v.at[pl.ds(0, hr), :], out_hbm.at[b, pl.ds(r0, hr), :], ysem)
        ycopy0.start()
        xcopy1.wait()
        plsc.parallel_loop(0, hr2 * qpr, unroll=4)(make_body(hr))
        ycopy0.wait()
        pltpu.sync_copy(y_v.at[pl.ds(hr, hr2), :],
                        out_hbm.at[b, pl.ds(r0 + hr, hr2), :])

    return pl.kernel(
        _sc_body,
        out_type=jax.ShapeDtypeStruct((b_dim, r_dim, c_dim), jnp.float32),
        mesh=mesh,
        scratch_types=[
            pltpu.VMEM((G,), jnp.float32),
            pltpu.VMEM((rows_w, c_dim), jnp.float32),
            pltpu.VMEM((rows_w, c_dim), jnp.float32),
            pltpu.SemaphoreType.DMA,
            pltpu.SemaphoreType.DMA,
            pltpu.SemaphoreType.DMA,
        ],
        compiler_params=pltpu.CompilerParams(needs_layout_passes=False),
    )


def kernel(x, levels, beta):
    beta_arr = jnp.reshape(beta, (1,)).astype(jnp.float32)
    tab2d = pl.pallas_call(
        _tab_body,
        out_shape=jax.ShapeDtypeStruct((GR, 128), jnp.float32),
        in_specs=[pl.BlockSpec(memory_space=pltpu.SMEM),
                  pl.BlockSpec(memory_space=pltpu.SMEM)],
        out_specs=pl.BlockSpec(memory_space=pltpu.VMEM),
    )(levels, beta_arr)
    tab = tab2d.reshape(G)

    xt = jnp.transpose(x, (0, 2, 1))
    b_dim, r_dim, c_dim = xt.shape
    yt = _make_sc_lookup(b_dim, r_dim, c_dim)(xt, tab)
    return jnp.transpose(yt, (0, 2, 1))


# speedup vs baseline: 1.0574x; 1.0574x over previous
import jax
import jax.numpy as jnp
from jax import lax
from jax.experimental import pallas as pl
from jax.experimental.pallas import tpu as pltpu
from jax.experimental.pallas import tpu_sc as plsc

K = 256
G = 2048
X0 = -8.0
X1 = 8.0
H = (X1 - X0) / (G - 1)
INV_H = 1.0 / H
GR = G // 128
KU = 15

NC, NS, L = 2, 16, 16
NW = NC * NS


def _tab_body(lev_ref, beta_ref, tab_ref):
    beta = beta_ref[0]
    l0 = lev_ref[0]
    gidx = (lax.broadcasted_iota(jnp.int32, (GR, 128), 0) * 128
            + lax.broadcasted_iota(jnp.int32, (GR, 128), 1))
    xg = X0 + H * gidx.astype(jnp.float32)

    def body(j, acc):
        for u in range(KU):
            k = j * KU + u
            lk = lev_ref[k]
            lk1 = lev_ref[k + 1]
            w = lk1 - lk
            b = 0.5 * (lk1 + lk)
            acc = acc + w * jax.nn.sigmoid(beta * (xg - b))
        return acc

    init = jnp.full((GR, 128), l0, jnp.float32)
    tab_ref[...] = lax.fori_loop(0, (K - 1) // KU, body, init)


def _make_sc_lookup(b_dim, r_dim, c_dim):
    rows_w = (b_dim * r_dim) // NW
    cv = c_dim // L
    VU = 4
    qpr = cv // VU
    w_per_b = r_dim // rows_w
    mesh = plsc.VectorSubcoreMesh(core_axis_name="c", subcore_axis_name="s",
                                  num_cores=NC, num_subcores=NS)
    hr = (rows_w // 2 + 7) // 8 * 8
    hr2 = rows_w - hr

    def _sc_body(x_hbm, tab_hbm, out_hbm, tab_v, x_v, y_v, shared_tab,
                 xsem0, xsem1, ysem):
        sid = lax.axis_index("s")
        wid = lax.axis_index("c") * NS + sid
        b = wid // w_per_b
        r0 = (wid % w_per_b) * rows_w

        xcopy0 = pltpu.make_async_copy(
            x_hbm.at[b, pl.ds(r0, hr), :], x_v.at[pl.ds(0, hr), :], xsem0)
        xcopy0.start()
        xcopy1 = pltpu.make_async_copy(
            x_hbm.at[b, pl.ds(r0 + hr, hr2), :], x_v.at[pl.ds(hr, hr2), :], xsem1)
        xcopy1.start()

        @pl.when(sid == 0)
        def _():
            pltpu.sync_copy(tab_hbm, shared_tab)
        plsc.subcore_barrier()
        pltpu.sync_copy(shared_tab, tab_v)

        def make_body(lo):
            def body(q):
                r = lo + q // qpr
                c0 = (q % qpr) * (VU * L)
                for j in range(VU):
                    o = c0 + j * L
                    xv = x_v[r, pl.ds(o, L)]
                    t = (jnp.clip(xv, X0, X1) - X0) * INV_H
                    idx = jnp.minimum(t.astype(jnp.int32), G - 2)
                    fr = t - idx.astype(jnp.float32)
                    y0 = plsc.load_gather(tab_v, [idx])
                    y1 = plsc.load_gather(tab_v, [idx + 1])
                    y_v[r, pl.ds(o, L)] = y0 + fr * (y1 - y0)
            return body

        xcopy0.wait()
        plsc.parallel_loop(0, hr * qpr, unroll=2)(make_body(0))
        ycopy0 = pltpu.make_async_copy(
            y_v.at[pl.ds(0, hr), :], out_hbm.at[b, pl.ds(r0, hr), :], ysem)
        ycopy0.start()
        xcopy1.wait()
        plsc.parallel_loop(0, hr2 * qpr, unroll=2)(make_body(hr))
        ycopy0.wait()
        pltpu.sync_copy(y_v.at[pl.ds(hr, hr2), :],
                        out_hbm.at[b, pl.ds(r0 + hr, hr2), :])

    return pl.kernel(
        _sc_body,
        out_type=jax.ShapeDtypeStruct((b_dim, r_dim, c_dim), jnp.float32),
        mesh=mesh,
        scratch_types=[
            pltpu.VMEM((G,), jnp.float32),
            pltpu.VMEM((rows_w, c_dim), jnp.float32),
            pltpu.VMEM((rows_w, c_dim), jnp.float32),
            pltpu.VMEM_SHARED((G,), jnp.float32),
            pltpu.SemaphoreType.DMA,
            pltpu.SemaphoreType.DMA,
            pltpu.SemaphoreType.DMA,
        ],
        compiler_params=pltpu.CompilerParams(needs_layout_passes=False),
    )


def kernel(x, levels, beta):
    beta_arr = jnp.reshape(beta, (1,)).astype(jnp.float32)
    tab2d = pl.pallas_call(
        _tab_body,
        out_shape=jax.ShapeDtypeStruct((GR, 128), jnp.float32),
        in_specs=[pl.BlockSpec(memory_space=pltpu.SMEM),
                  pl.BlockSpec(memory_space=pltpu.SMEM)],
        out_specs=pl.BlockSpec(memory_space=pltpu.VMEM),
    )(levels, beta_arr)
    tab = tab2d.reshape(G)

    xt = jnp.transpose(x, (0, 2, 1))
    b_dim, r_dim, c_dim = xt.shape
    yt = _make_sc_lookup(b_dim, r_dim, c_dim)(xt, tab)
    return jnp.transpose(yt, (0, 2, 1))
